# Initial kernel scaffold; baseline (speedup 1.0000x reference)
import jax
import jax.numpy as jnp
from jax.experimental import pallas as pl


def kernel(features, edge_index, W1_0, b1_0, W2_0, b2_0, W1_1, b1_1, W2_1, b2_1):
    raise NotImplementedError("write your pallas kernel here")


# speedup vs baseline: 10.8754x; 10.8754x over previous
import functools

import jax
import jax.numpy as jnp
from jax import lax
from jax.experimental import pallas as pl
from jax.experimental.pallas import tpu as pltpu
from jax.experimental.pallas import tpu_sc as plsc

N = 10000
E = 320000
D = 128

NC = 2
NS = 16
L = 16
NW = NC * NS
EW = E // NW
C = 80
NCH = EW // C
NA = 10240
DUMMY = N
ROWS_W = NA // NS

_mesh = None


def _get_mesh():
    global _mesh
    if _mesh is None:
        _mesh = plsc.VectorSubcoreMesh(core_axis_name="c", subcore_axis_name="s")
    return _mesh


def _deg_body(src_hbm, dst_hbm, zt_hbm, degp_hbm, dstp_hbm,
              srcv, dstv, dst16v, ones, deg_sh):
    c = lax.axis_index("c")
    s = lax.axis_index("s")
    wid = c * NS + s
    base_e = wid * EW

    for g in range(C // L):
        ones[pl.ds(g * L, L)] = jnp.full((L,), 1.0, jnp.float32)
    pltpu.sync_copy(zt_hbm, deg_sh.at[pl.ds(s * ROWS_W * L, ROWS_W * L)])
    plsc.subcore_barrier()

    def chunk(k, _):
        b = base_e + k * C
        pltpu.sync_copy(src_hbm.at[pl.ds(b, C)], srcv)
        pltpu.sync_copy(dst_hbm.at[pl.ds(b, C)], dstv)
        for g in range(C // L):
            sv = srcv[pl.ds(g * L, L)]
            dv = dstv[pl.ds(g * L, L)]
            dm = jnp.where(sv != dv, dv, jnp.full((L,), DUMMY, jnp.int32))
            dstv[pl.ds(g * L, L)] = dm
            dst16v[pl.ds(g * L, L)] = dm * L
        pltpu.sync_copy(ones, deg_sh.at[dst16v], add=True)
        pltpu.sync_copy(dstv, dstp_hbm.at[pl.ds(b, C)])
        return _

    lax.fori_loop(0, NCH, chunk, None)
    plsc.subcore_barrier()
    pltpu.sync_copy(deg_sh.at[pl.ds(s * ROWS_W * L, ROWS_W * L)],
                    degp_hbm.at[pl.ds((c * NA + s * ROWS_W) * L, ROWS_W * L)])


def _sc_deg(src, dst, zt):
    return pl.kernel(
        _deg_body,
        out_type=(
            jax.ShapeDtypeStruct((NC * NA * L,), jnp.float32),
            jax.ShapeDtypeStruct((E,), jnp.int32),
        ),
        mesh=_get_mesh(),
        scratch_types=(
            pltpu.VMEM((C,), jnp.int32),
            pltpu.VMEM((C,), jnp.int32),
            pltpu.VMEM((C,), jnp.int32),
            pltpu.VMEM((C,), jnp.float32),
            pltpu.VMEM_SHARED((NA * L,), jnp.float32),
        ),
    )(src, dst, zt)


def _gs_body(with_t, g_hbm, src_hbm, dstp_hbm, r_hbm, zc_hbm, zt_hbm,
             ap_hbm, tp_hbm, srcv, dstv, dst16v, rvals, grows, a_sh, t_sh,
             gsem, tsem):
    c = lax.axis_index("c")
    s = lax.axis_index("s")
    wid = c * NS + s
    base_e = wid * EW

    if with_t:
        pltpu.sync_copy(zt_hbm, t_sh.at[pl.ds(s * ROWS_W * L, ROWS_W * L)])
    pltpu.sync_copy(zc_hbm, a_sh.at[pl.ds(s * ROWS_W, ROWS_W)])
    plsc.subcore_barrier()

    def chunk(k, _):
        b = base_e + k * C
        pltpu.sync_copy(src_hbm.at[pl.ds(b, C)], srcv)
        pltpu.sync_copy(dstp_hbm.at[pl.ds(b, C)], dstv)
        cp = pltpu.async_copy(g_hbm.at[srcv], grows, gsem)
        if with_t:
            tp_ = pltpu.async_copy(r_hbm.at[srcv], rvals, tsem)
            for g in range(C // L):
                dst16v[pl.ds(g * L, L)] = dstv[pl.ds(g * L, L)] * L
            tp_.wait()
            pltpu.sync_copy(rvals, t_sh.at[dst16v], add=True)
        cp.wait()
        pltpu.sync_copy(grows, a_sh.at[dstv], add=True)
        return _

    lax.fori_loop(0, NCH, chunk, None)
    plsc.subcore_barrier()
    pltpu.sync_copy(a_sh.at[pl.ds(s * ROWS_W, ROWS_W)],
                    ap_hbm.at[pl.ds(c * NA + s * ROWS_W, ROWS_W)])
    if with_t:
        pltpu.sync_copy(t_sh.at[pl.ds(s * ROWS_W * L, ROWS_W * L)],
                        tp_hbm.at[pl.ds((c * NA + s * ROWS_W) * L, ROWS_W * L)])


def _sc_gs(gtab, src, dstp, rflat, zc, zt, with_t):
    out_type = [jax.ShapeDtypeStruct((NC * NA, D), jnp.float32)]
    if with_t:
        out_type.append(jax.ShapeDtypeStruct((NC * NA * L,), jnp.float32))
    else:
        out_type.append(jax.ShapeDtypeStruct((8 * L,), jnp.float32))
    return pl.kernel(
        functools.partial(_gs_body, with_t),
        out_type=tuple(out_type),
        mesh=_get_mesh(),
        scratch_types=(
            pltpu.VMEM((C,), jnp.int32),
            pltpu.VMEM((C,), jnp.int32),
            pltpu.VMEM((C,), jnp.int32),
            pltpu.VMEM((C,), jnp.float32),
            pltpu.VMEM((C, D), jnp.float32),
            pltpu.VMEM_SHARED((NA, D), jnp.float32),
            pltpu.VMEM_SHARED((NA * L,), jnp.float32),
            pltpu.SemaphoreType.DMA,
            pltpu.SemaphoreType.DMA,
        ),
    )(gtab, src, dstp, rflat, zc, zt)


BN = 256
GRID = NA // BN


def _prep_kernel(dega_ref, degb_ref, feat_ref, g0_ref, r16_ref):
    deg = (jnp.sum(dega_ref[...], axis=1, keepdims=True)
           + jnp.sum(degb_ref[...], axis=1, keepdims=True))
    r = lax.rsqrt(jnp.maximum(deg, 1.0))
    g0_ref[...] = r * feat_ref[...]
    cols = lax.broadcasted_iota(jnp.int32, (BN, L), 1)
    r16_ref[...] = jnp.where(cols == 0, r, 0.0)


def _tc_prep(degp, featp):
    return pl.pallas_call(
        _prep_kernel,
        grid=(GRID,),
        in_specs=[
            pl.BlockSpec((BN, L), lambda i: (i, 0)),
            pl.BlockSpec((BN, L), lambda i: (i + GRID, 0)),
            pl.BlockSpec((BN, D), lambda i: (i, 0)),
        ],
        out_specs=[
            pl.BlockSpec((BN, D), lambda i: (i, 0)),
            pl.BlockSpec((BN, L), lambda i: (i, 0)),
        ],
        out_shape=[
            jax.ShapeDtypeStruct((NA, D), jnp.float32),
            jax.ShapeDtypeStruct((NA, L), jnp.float32),
        ],
    )(degp, degp, featp)


def _dense_kernel(leaky, with_g, aa_ref, ab_ref, ta_ref, tb_ref, r16_ref,
                  h_ref, w1_ref, b1_ref, w2_ref, b2_ref, *out_refs):
    r = jnp.sum(r16_ref[...], axis=1, keepdims=True)
    t = (jnp.sum(ta_ref[...], axis=1, keepdims=True)
         + jnp.sum(tb_ref[...], axis=1, keepdims=True))
    a = r * (aa_ref[...] + ab_ref[...])
    s = r * t
    h = h_ref[...]
    b1 = b1_ref[...]
    b2 = b2_ref[...]
    out = (jnp.dot(a + h, w1_ref[...], preferred_element_type=jnp.float32)
           + jnp.dot(a * h, w2_ref[...], preferred_element_type=jnp.float32)
           + b1 + s * (b1 + b2))
    if leaky:
        out = jnp.where(out > 0, out, 0.01 * out)
    out_refs[0][...] = out
    if with_g:
        out_refs[1][...] = r * out


def _tc_dense(ap, tp, r16, h, w1, b1, w2, b2, leaky, with_g):
    out_specs = [pl.BlockSpec((BN, D), lambda i: (i, 0))]
    out_shape = [jax.ShapeDtypeStruct((NA, D), jnp.float32)]
    if with_g:
        out_specs.append(pl.BlockSpec((BN, D), lambda i: (i, 0)))
        out_shape.append(jax.ShapeDtypeStruct((NA, D), jnp.float32))
    return pl.pallas_call(
        functools.partial(_dense_kernel, leaky, with_g),
        grid=(GRID,),
        in_specs=[
            pl.BlockSpec((BN, D), lambda i: (i, 0)),
            pl.BlockSpec((BN, D), lambda i: (i + GRID, 0)),
            pl.BlockSpec((BN, L), lambda i: (i, 0)),
            pl.BlockSpec((BN, L), lambda i: (i + GRID, 0)),
            pl.BlockSpec((BN, L), lambda i: (i, 0)),
            pl.BlockSpec((BN, D), lambda i: (i, 0)),
            pl.BlockSpec((D, D), lambda i: (0, 0)),
            pl.BlockSpec((1, D), lambda i: (0, 0)),
            pl.BlockSpec((D, D), lambda i: (0, 0)),
            pl.BlockSpec((1, D), lambda i: (0, 0)),
        ],
        out_specs=out_specs,
        out_shape=out_shape,
    )(ap, ap, tp, tp, r16, h, w1, b1, w2, b2)


def kernel(features, edge_index, W1_0, b1_0, W2_0, b2_0, W1_1, b1_1, W2_1, b2_1):
    src = edge_index[0]
    dst = edge_index[1]
    featp = jnp.pad(features, ((0, NA - N), (0, 0)))
    zc = jnp.zeros((ROWS_W, D), jnp.float32)
    zt = jnp.zeros((ROWS_W * L,), jnp.float32)

    degp, dstp = _sc_deg(src, dst, zt)
    g0, r16 = _tc_prep(degp.reshape(NC * NA, L), featp)
    rflat = r16[:, 0]

    ap0, tp = _sc_gs(g0, src, dstp, rflat, zc, zt, with_t=True)
    tp = tp.reshape(NC * NA, L)
    h1, g1 = _tc_dense(ap0, tp, r16, featp,
                       W1_0, b1_0.reshape(1, D), W2_0, b2_0.reshape(1, D),
                       leaky=True, with_g=True)
    ap1, _unused = _sc_gs(g1, src, dstp, rflat, zc, zt, with_t=False)
    (h2,) = _tc_dense(ap1, tp, r16, h1,
                      W1_1, b1_1.reshape(1, D), W2_1, b2_1.reshape(1, D),
                      leaky=False, with_g=False)

    return jnp.concatenate([features, h1[:N], h2[:N]], axis=-1)


# speedup vs baseline: 19.7846x; 1.8192x over previous
import functools

import jax
import jax.numpy as jnp
from jax import lax
from jax.experimental import pallas as pl
from jax.experimental.pallas import tpu as pltpu
from jax.experimental.pallas import tpu_sc as plsc

N = 10000
E = 320000
D = 128

NC = 2
NS = 16
L = 16
L2 = 8
NW = NC * NS
EW = E // NW
C = 80
NCH = EW // C
NB = 5
NBG = 4
NA = 10240
DUMMY = N
ROWS_W = NA // NS

_mesh = None


def _get_mesh():
    global _mesh
    if _mesh is None:
        _mesh = plsc.VectorSubcoreMesh(core_axis_name="c", subcore_axis_name="s")
    return _mesh


def _deg_body(src_hbm, dst_hbm, zt_hbm, degp_hbm, dstp_hbm,
              srcv, dstv, dst16v, ones, deg_sh, lsems, ldsems, ssems, osems):
    c = lax.axis_index("c")
    s = lax.axis_index("s")
    wid = c * NS + s
    base_e = wid * EW

    for g in range(C // L):
        ones[pl.ds(g * L, L)] = jnp.full((L,), 1.0, jnp.float32)
    pltpu.sync_copy(zt_hbm, deg_sh.at[pl.ds(s * ROWS_W * L2, ROWS_W * L2)])
    plsc.subcore_barrier()

    def rounds(i, _):
        k0 = i * NB
        lds = []
        for b in range(NB):
            off = base_e + (k0 + b) * C
            lds.append((
                pltpu.async_copy(src_hbm.at[pl.ds(off, C)], srcv[b], lsems[b]),
                pltpu.async_copy(dst_hbm.at[pl.ds(off, C)], dstv[b], ldsems[b]),
            ))
        sds = []
        for b in range(NB):
            off = base_e + (k0 + b) * C
            lds[b][0].wait()
            lds[b][1].wait()
            for g in range(C // L):
                sv = srcv[b][pl.ds(g * L, L)]
                dv = dstv[b][pl.ds(g * L, L)]
                dm = jnp.where(sv != dv, dv, jnp.full((L,), DUMMY, jnp.int32))
                dstv[b][pl.ds(g * L, L)] = dm
                dst16v[b][pl.ds(g * L, L)] = dm * L2
            sds.append(pltpu.async_copy(ones, deg_sh.at[dst16v[b]], ssems[b],
                                        add=True))
            sds.append(pltpu.async_copy(dstv[b], dstp_hbm.at[pl.ds(off, C)],
                                        osems[b]))
        for sd in sds:
            sd.wait()
        return _

    lax.fori_loop(0, NCH // NB, rounds, None)
    plsc.subcore_barrier()
    pltpu.sync_copy(deg_sh.at[pl.ds(s * ROWS_W * L2, ROWS_W * L2)],
                    degp_hbm.at[pl.ds((c * NA + s * ROWS_W) * L2, ROWS_W * L2)])


def _sc_deg(src, dst, zt):
    def body(src_hbm, dst_hbm, zt_hbm, degp_hbm, dstp_hbm, *scr):
        srcv = list(scr[0:NB])
        dstv = list(scr[NB:2 * NB])
        dst16v = list(scr[2 * NB:3 * NB])
        ones, deg_sh = scr[3 * NB], scr[3 * NB + 1]
        sems = list(scr[3 * NB + 2:])
        lsems, ldsems = sems[0:NB], sems[NB:2 * NB]
        ssems, osems = sems[2 * NB:3 * NB], sems[3 * NB:4 * NB]
        _deg_body(src_hbm, dst_hbm, zt_hbm, degp_hbm, dstp_hbm,
                  srcv, dstv, dst16v, ones, deg_sh, lsems, ldsems, ssems, osems)

    scratch = (
        [pltpu.VMEM((C,), jnp.int32) for _ in range(3 * NB)]
        + [pltpu.VMEM((C,), jnp.float32),
           pltpu.VMEM_SHARED((NA * L2,), jnp.float32)]
        + [pltpu.SemaphoreType.DMA for _ in range(4 * NB)]
    )
    return pl.kernel(
        body,
        out_type=(
            jax.ShapeDtypeStruct((NC * NA * L2,), jnp.float32),
            jax.ShapeDtypeStruct((E,), jnp.int32),
        ),
        mesh=_get_mesh(),
        scratch_types=tuple(scratch),
    )(src, dst, zt)


def _gs_body(with_t, g_hbm, src_hbm, dstp_hbm, r_hbm, zc_hbm, zt_hbm,
             ap_hbm, tp_hbm, srcv, dstv, dst16v, rvals, grows, a_sh, t_sh,
             lsems, ldsems, gsems, rsems, ssems, tsems):
    c = lax.axis_index("c")
    s = lax.axis_index("s")
    wid = c * NS + s
    base_e = wid * EW

    if with_t:
        pltpu.sync_copy(zt_hbm, t_sh.at[pl.ds(s * ROWS_W * L2, ROWS_W * L2)])
    pltpu.sync_copy(zc_hbm, a_sh.at[pl.ds(s * ROWS_W, ROWS_W)])
    plsc.subcore_barrier()

    def run_batch(k0, nb):
        lds = []
        for b in range(nb):
            off = base_e + (k0 + b) * C
            lds.append((
                pltpu.async_copy(src_hbm.at[pl.ds(off, C)], srcv[b], lsems[b]),
                pltpu.async_copy(dstp_hbm.at[pl.ds(off, C)], dstv[b], ldsems[b]),
            ))
        gds = []
        for b in range(nb):
            lds[b][0].wait()
            lds[b][1].wait()
            gd = [pltpu.async_copy(g_hbm.at[srcv[b]], grows[b], gsems[b])]
            if with_t:
                gd.append(pltpu.async_copy(r_hbm.at[srcv[b]], rvals[b], rsems[b]))
                for g in range(C // L):
                    dst16v[b][pl.ds(g * L, L)] = dstv[b][pl.ds(g * L, L)] * L2
            gds.append(gd)
        sds = []
        for b in range(nb):
            for gd in gds[b]:
                gd.wait()
            sds.append(pltpu.async_copy(grows[b], a_sh.at[dstv[b]], ssems[b],
                                        add=True))
            if with_t:
                sds.append(pltpu.async_copy(rvals[b], t_sh.at[dst16v[b]],
                                            tsems[b], add=True))
        for sd in sds:
            sd.wait()

    def rounds(i, _):
        run_batch(i * NBG, NBG)
        return _

    lax.fori_loop(0, NCH // NBG, rounds, None)
    for kt in range(NCH - (NCH // NBG) * NBG):
        run_batch((NCH // NBG) * NBG + kt, 1)
    plsc.subcore_barrier()
    pltpu.sync_copy(a_sh.at[pl.ds(s * ROWS_W, ROWS_W)],
                    ap_hbm.at[pl.ds(c * NA + s * ROWS_W, ROWS_W)])
    if with_t:
        pltpu.sync_copy(t_sh.at[pl.ds(s * ROWS_W * L2, ROWS_W * L2)],
                        tp_hbm.at[pl.ds((c * NA + s * ROWS_W) * L2, ROWS_W * L2)])


def _sc_gs(gtab, src, dstp, rflat, zc, zt, with_t):
    out_type = [jax.ShapeDtypeStruct((NC * NA, D), jnp.float32)]
    if with_t:
        out_type.append(jax.ShapeDtypeStruct((NC * NA * L2,), jnp.float32))
    else:
        out_type.append(jax.ShapeDtypeStruct((8 * L,), jnp.float32))

    def body(g_hbm, src_hbm, dstp_hbm, r_hbm, zc_hbm, zt_hbm,
             ap_hbm, tp_hbm, *scr):
        srcv = list(scr[0:NBG])
        dstv = list(scr[NBG:2 * NBG])
        dst16v = list(scr[2 * NBG:3 * NBG])
        rvals = list(scr[3 * NBG:4 * NBG])
        grows = list(scr[4 * NBG:5 * NBG])
        a_sh, t_sh = scr[5 * NBG], scr[5 * NBG + 1]
        sems = list(scr[5 * NBG + 2:])
        lsems, ldsems = sems[0:NBG], sems[NBG:2 * NBG]
        gsems, rsems = sems[2 * NBG:3 * NBG], sems[3 * NBG:4 * NBG]
        ssems, tsems = sems[4 * NBG:5 * NBG], sems[5 * NBG:6 * NBG]
        _gs_body(with_t, g_hbm, src_hbm, dstp_hbm, r_hbm, zc_hbm, zt_hbm,
                 ap_hbm, tp_hbm, srcv, dstv, dst16v, rvals, grows, a_sh, t_sh,
                 lsems, ldsems, gsems, rsems, ssems, tsems)

    scratch = (
        [pltpu.VMEM((C,), jnp.int32) for _ in range(NBG)]
        + [pltpu.VMEM((C,), jnp.int32) for _ in range(NBG)]
        + [pltpu.VMEM((C,), jnp.int32) for _ in range(NBG)]
        + [pltpu.VMEM((C,), jnp.float32) for _ in range(NBG)]
        + [pltpu.VMEM((C, D), jnp.float32) for _ in range(NBG)]
        + [pltpu.VMEM_SHARED((NA, D), jnp.float32),
           pltpu.VMEM_SHARED((NA * L2,), jnp.float32)]
        + [pltpu.SemaphoreType.DMA for _ in range(6 * NBG)]
    )
    return pl.kernel(
        body,
        out_type=tuple(out_type),
        mesh=_get_mesh(),
        scratch_types=tuple(scratch),
    )(gtab, src, dstp, rflat, zc, zt)


BN = 256
GRID = NA // BN


def _prep_kernel(dega_ref, degb_ref, feat_ref, g0_ref, r16_ref):
    deg = (jnp.sum(dega_ref[...], axis=1, keepdims=True)
           + jnp.sum(degb_ref[...], axis=1, keepdims=True))
    r = lax.rsqrt(jnp.maximum(deg, 1.0))
    g0_ref[...] = r * feat_ref[...]
    cols = lax.broadcasted_iota(jnp.int32, (BN, L), 1)
    r16_ref[...] = jnp.where(cols == 0, r, 0.0)


def _tc_prep(degp, featp):
    return pl.pallas_call(
        _prep_kernel,
        grid=(GRID,),
        in_specs=[
            pl.BlockSpec((BN, L2), lambda i: (i, 0)),
            pl.BlockSpec((BN, L2), lambda i: (i + GRID, 0)),
            pl.BlockSpec((BN, D), lambda i: (i, 0)),
        ],
        out_specs=[
            pl.BlockSpec((BN, D), lambda i: (i, 0)),
            pl.BlockSpec((BN, L), lambda i: (i, 0)),
        ],
        out_shape=[
            jax.ShapeDtypeStruct((NA, D), jnp.float32),
            jax.ShapeDtypeStruct((NA, L), jnp.float32),
        ],
    )(degp, degp, featp)


def _dense_kernel(leaky, with_g, aa_ref, ab_ref, ta_ref, tb_ref, r16_ref,
                  h_ref, w1_ref, b1_ref, w2_ref, b2_ref, *out_refs):
    r = jnp.sum(r16_ref[...], axis=1, keepdims=True)
    t = (jnp.sum(ta_ref[...], axis=1, keepdims=True)
         + jnp.sum(tb_ref[...], axis=1, keepdims=True))
    a = r * (aa_ref[...] + ab_ref[...])
    s = r * t
    h = h_ref[...]
    b1 = b1_ref[...]
    b2 = b2_ref[...]
    out = (jnp.dot(a + h, w1_ref[...], preferred_element_type=jnp.float32)
           + jnp.dot(a * h, w2_ref[...], preferred_element_type=jnp.float32)
           + b1 + s * (b1 + b2))
    if leaky:
        out = jnp.where(out > 0, out, 0.01 * out)
    out_refs[0][...] = out
    if with_g:
        out_refs[1][...] = r * out


def _tc_dense(ap, tp, r16, h, w1, b1, w2, b2, leaky, with_g):
    out_specs = [pl.BlockSpec((BN, D), lambda i: (i, 0))]
    out_shape = [jax.ShapeDtypeStruct((NA, D), jnp.float32)]
    if with_g:
        out_specs.append(pl.BlockSpec((BN, D), lambda i: (i, 0)))
        out_shape.append(jax.ShapeDtypeStruct((NA, D), jnp.float32))
    return pl.pallas_call(
        functools.partial(_dense_kernel, leaky, with_g),
        grid=(GRID,),
        in_specs=[
            pl.BlockSpec((BN, D), lambda i: (i, 0)),
            pl.BlockSpec((BN, D), lambda i: (i + GRID, 0)),
            pl.BlockSpec((BN, L2), lambda i: (i, 0)),
            pl.BlockSpec((BN, L2), lambda i: (i + GRID, 0)),
            pl.BlockSpec((BN, L), lambda i: (i, 0)),
            pl.BlockSpec((BN, D), lambda i: (i, 0)),
            pl.BlockSpec((D, D), lambda i: (0, 0)),
            pl.BlockSpec((1, D), lambda i: (0, 0)),
            pl.BlockSpec((D, D), lambda i: (0, 0)),
            pl.BlockSpec((1, D), lambda i: (0, 0)),
        ],
        out_specs=out_specs,
        out_shape=out_shape,
    )(ap, ap, tp, tp, r16, h, w1, b1, w2, b2)


def kernel(features, edge_index, W1_0, b1_0, W2_0, b2_0, W1_1, b1_1, W2_1, b2_1):
    src = edge_index[0]
    dst = edge_index[1]
    featp = jnp.pad(features, ((0, NA - N), (0, 0)))
    zc = jnp.zeros((ROWS_W, D), jnp.float32)
    zt = jnp.zeros((ROWS_W * L2,), jnp.float32)

    degp, dstp = _sc_deg(src, dst, zt)
    g0, r16 = _tc_prep(degp.reshape(NC * NA, L2), featp)
    rflat = r16[:, 0]

    ap0, tp = _sc_gs(g0, src, dstp, rflat, zc, zt, with_t=True)
    tp = tp.reshape(NC * NA, L2)
    h1, g1 = _tc_dense(ap0, tp, r16, featp,
                       W1_0, b1_0.reshape(1, D), W2_0, b2_0.reshape(1, D),
                       leaky=True, with_g=True)
    ap1, _unused = _sc_gs(g1, src, dstp, rflat, zc, zt, with_t=False)
    (h2,) = _tc_dense(ap1, tp, r16, h1,
                      W1_1, b1_1.reshape(1, D), W2_1, b2_1.reshape(1, D),
                      leaky=False, with_g=False)

    return jnp.concatenate([features, h1[:N], h2[:N]], axis=-1)


# speedup vs baseline: 20.2236x; 1.0222x over previous
import functools

import jax
import jax.numpy as jnp
from jax import lax
from jax.experimental import pallas as pl
from jax.experimental.pallas import tpu as pltpu
from jax.experimental.pallas import tpu_sc as plsc

N = 10000
E = 320000
D = 128

NC = 2
NS = 16
L = 16
L2 = 8
NW = NC * NS
EW = E // NW
C = 80
NCH = EW // C
NB = 5
NBG = 4
NA = 10240
DUMMY = N
ROWS_W = NA // NS

_mesh = None


def _get_mesh():
    global _mesh
    if _mesh is None:
        _mesh = plsc.VectorSubcoreMesh(core_axis_name="c", subcore_axis_name="s")
    return _mesh


def _deg_body(src_hbm, dst_hbm, zt_hbm, degp_hbm, dstp_hbm,
              srcv, dstv, dst16v, ones, deg_sh, lsems, ldsems, ssems, osems):
    c = lax.axis_index("c")
    s = lax.axis_index("s")
    wid = c * NS + s
    base_e = wid * EW

    for g in range(C // L):
        ones[pl.ds(g * L, L)] = jnp.full((L,), 1.0, jnp.float32)
    pltpu.sync_copy(zt_hbm, deg_sh.at[pl.ds(s * ROWS_W * L2, ROWS_W * L2)])
    plsc.subcore_barrier()

    def rounds(i, _):
        k0 = i * NB
        lds = []
        for b in range(NB):
            off = base_e + (k0 + b) * C
            lds.append((
                pltpu.async_copy(src_hbm.at[pl.ds(off, C)], srcv[b], lsems[b]),
                pltpu.async_copy(dst_hbm.at[pl.ds(off, C)], dstv[b], ldsems[b]),
            ))
        sds = []
        for b in range(NB):
            off = base_e + (k0 + b) * C
            lds[b][0].wait()
            lds[b][1].wait()
            for g in range(C // L):
                sv = srcv[b][pl.ds(g * L, L)]
                dv = dstv[b][pl.ds(g * L, L)]
                dm = jnp.where(sv != dv, dv, jnp.full((L,), DUMMY, jnp.int32))
                dstv[b][pl.ds(g * L, L)] = dm
                dst16v[b][pl.ds(g * L, L)] = dm * L2
            sds.append(pltpu.async_copy(ones, deg_sh.at[dst16v[b]], ssems[b],
                                        add=True))
            sds.append(pltpu.async_copy(dstv[b], dstp_hbm.at[pl.ds(off, C)],
                                        osems[b]))
        for sd in sds:
            sd.wait()
        return _

    lax.fori_loop(0, NCH // NB, rounds, None)
    plsc.subcore_barrier()
    pltpu.sync_copy(deg_sh.at[pl.ds(s * ROWS_W * L2, ROWS_W * L2)],
                    degp_hbm.at[pl.ds((c * NA + s * ROWS_W) * L2, ROWS_W * L2)])


def _sc_deg(src, dst, zt):
    def body(src_hbm, dst_hbm, zt_hbm, degp_hbm, dstp_hbm, *scr):
        srcv = list(scr[0:NB])
        dstv = list(scr[NB:2 * NB])
        dst16v = list(scr[2 * NB:3 * NB])
        ones, deg_sh = scr[3 * NB], scr[3 * NB + 1]
        sems = list(scr[3 * NB + 2:])
        lsems, ldsems = sems[0:NB], sems[NB:2 * NB]
        ssems, osems = sems[2 * NB:3 * NB], sems[3 * NB:4 * NB]
        _deg_body(src_hbm, dst_hbm, zt_hbm, degp_hbm, dstp_hbm,
                  srcv, dstv, dst16v, ones, deg_sh, lsems, ldsems, ssems, osems)

    scratch = (
        [pltpu.VMEM((C,), jnp.int32) for _ in range(3 * NB)]
        + [pltpu.VMEM((C,), jnp.float32),
           pltpu.VMEM_SHARED((NA * L2,), jnp.float32)]
        + [pltpu.SemaphoreType.DMA for _ in range(4 * NB)]
    )
    return pl.kernel(
        body,
        out_type=(
            jax.ShapeDtypeStruct((NC * NA * L2,), jnp.float32),
            jax.ShapeDtypeStruct((E,), jnp.int32),
        ),
        mesh=_get_mesh(),
        scratch_types=tuple(scratch),
    )(src, dst, zt)


def _gs_body(with_t, g_hbm, src_hbm, dstp_hbm, r_hbm, zc_hbm, zt_hbm,
             ap_hbm, tp_hbm, srcv, dstv, dst16v, rvals, grows, a_sh, t_sh,
             lsems, ldsems, gsems, rsems, ssems, tsems):
    c = lax.axis_index("c")
    s = lax.axis_index("s")
    wid = c * NS + s
    base_e = wid * EW

    if with_t:
        pltpu.sync_copy(zt_hbm, t_sh.at[pl.ds(s * ROWS_W * L2, ROWS_W * L2)])
    pltpu.sync_copy(zc_hbm, a_sh.at[pl.ds(s * ROWS_W, ROWS_W)])
    plsc.subcore_barrier()

    def run_batch(k0, nb):
        lds = []
        for b in range(nb):
            off = base_e + (k0 + b) * C
            lds.append((
                pltpu.async_copy(src_hbm.at[pl.ds(off, C)], srcv[b], lsems[b]),
                pltpu.async_copy(dstp_hbm.at[pl.ds(off, C)], dstv[b], ldsems[b]),
            ))
        gds = []
        for b in range(nb):
            lds[b][0].wait()
            lds[b][1].wait()
            gd = [pltpu.async_copy(g_hbm.at[srcv[b]], grows[b], gsems[b])]
            if with_t:
                gd.append(pltpu.async_copy(r_hbm.at[srcv[b]], rvals[b], rsems[b]))
                for g in range(C // L):
                    dst16v[b][pl.ds(g * L, L)] = dstv[b][pl.ds(g * L, L)] * L2
            gds.append(gd)
        sds = []
        for b in range(nb):
            for gd in gds[b]:
                gd.wait()
            sds.append(pltpu.async_copy(grows[b], a_sh.at[dstv[b]], ssems[b],
                                        add=True))
            if with_t:
                sds.append(pltpu.async_copy(rvals[b], t_sh.at[dst16v[b]],
                                            tsems[b], add=True))
        for sd in sds:
            sd.wait()

    def rounds(i, _):
        run_batch(i * NBG, NBG)
        return _

    lax.fori_loop(0, NCH // NBG, rounds, None)
    for kt in range(NCH - (NCH // NBG) * NBG):
        run_batch((NCH // NBG) * NBG + kt, 1)
    plsc.subcore_barrier()
    pltpu.sync_copy(a_sh.at[pl.ds(s * ROWS_W, ROWS_W)],
                    ap_hbm.at[pl.ds(c * NA + s * ROWS_W, ROWS_W)])
    if with_t:
        pltpu.sync_copy(t_sh.at[pl.ds(s * ROWS_W * L2, ROWS_W * L2)],
                        tp_hbm.at[pl.ds((c * NA + s * ROWS_W) * L2, ROWS_W * L2)])


def _sc_gs(gtab, src, dstp, rflat, zc, zt, with_t):
    out_type = [jax.ShapeDtypeStruct((NC * NA, D), jnp.float32)]
    if with_t:
        out_type.append(jax.ShapeDtypeStruct((NC * NA * L2,), jnp.float32))
    else:
        out_type.append(jax.ShapeDtypeStruct((8 * L,), jnp.float32))

    def body(g_hbm, src_hbm, dstp_hbm, r_hbm, zc_hbm, zt_hbm,
             ap_hbm, tp_hbm, *scr):
        srcv = list(scr[0:NBG])
        dstv = list(scr[NBG:2 * NBG])
        dst16v = list(scr[2 * NBG:3 * NBG])
        rvals = list(scr[3 * NBG:4 * NBG])
        grows = list(scr[4 * NBG:5 * NBG])
        a_sh, t_sh = scr[5 * NBG], scr[5 * NBG + 1]
        sems = list(scr[5 * NBG + 2:])
        lsems, ldsems = sems[0:NBG], sems[NBG:2 * NBG]
        gsems, rsems = sems[2 * NBG:3 * NBG], sems[3 * NBG:4 * NBG]
        ssems, tsems = sems[4 * NBG:5 * NBG], sems[5 * NBG:6 * NBG]
        _gs_body(with_t, g_hbm, src_hbm, dstp_hbm, r_hbm, zc_hbm, zt_hbm,
                 ap_hbm, tp_hbm, srcv, dstv, dst16v, rvals, grows, a_sh, t_sh,
                 lsems, ldsems, gsems, rsems, ssems, tsems)

    scratch = (
        [pltpu.VMEM((C,), jnp.int32) for _ in range(NBG)]
        + [pltpu.VMEM((C,), jnp.int32) for _ in range(NBG)]
        + [pltpu.VMEM((C,), jnp.int32) for _ in range(NBG)]
        + [pltpu.VMEM((C,), jnp.float32) for _ in range(NBG)]
        + [pltpu.VMEM((C, D), jnp.float32) for _ in range(NBG)]
        + [pltpu.VMEM_SHARED((NA, D), jnp.float32),
           pltpu.VMEM_SHARED((NA * L2,), jnp.float32)]
        + [pltpu.SemaphoreType.DMA for _ in range(6 * NBG)]
    )
    return pl.kernel(
        body,
        out_type=tuple(out_type),
        mesh=_get_mesh(),
        scratch_types=tuple(scratch),
    )(gtab, src, dstp, rflat, zc, zt)


BN = 256
GRID = NA // BN


def _prep_kernel(dega_ref, degb_ref, feat_ref, g0_ref, r16_ref, o384_ref):
    deg = (jnp.sum(dega_ref[...], axis=1, keepdims=True)
           + jnp.sum(degb_ref[...], axis=1, keepdims=True))
    r = lax.rsqrt(jnp.maximum(deg, 1.0))
    f = feat_ref[...]
    g0_ref[...] = r * f
    cols = lax.broadcasted_iota(jnp.int32, (BN, L), 1)
    r16_ref[...] = jnp.where(cols == 0, r, 0.0)
    o384_ref[...] = f


def _tc_prep(degp, feat):
    return pl.pallas_call(
        _prep_kernel,
        grid=(GRID,),
        in_specs=[
            pl.BlockSpec((BN, L2), lambda i: (i, 0)),
            pl.BlockSpec((BN, L2), lambda i: (i + GRID, 0)),
            pl.BlockSpec((BN, D), lambda i: (i, 0)),
        ],
        out_specs=[
            pl.BlockSpec((BN, D), lambda i: (i, 0)),
            pl.BlockSpec((BN, L), lambda i: (i, 0)),
            pl.BlockSpec((BN, D), lambda i: (i, 0)),
        ],
        out_shape=[
            jax.ShapeDtypeStruct((NA, D), jnp.float32),
            jax.ShapeDtypeStruct((NA, L), jnp.float32),
            jax.ShapeDtypeStruct((N, 3 * D), jnp.float32),
        ],
    )(degp, degp, feat)


def _dense_kernel(leaky, with_g, aa_ref, ab_ref, ta_ref, tb_ref, r16_ref,
                  h_ref, w1_ref, b1_ref, w2_ref, b2_ref, o384_in_ref,
                  *out_refs):
    del o384_in_ref
    r = jnp.sum(r16_ref[...], axis=1, keepdims=True)
    t = (jnp.sum(ta_ref[...], axis=1, keepdims=True)
         + jnp.sum(tb_ref[...], axis=1, keepdims=True))
    a = r * (aa_ref[...] + ab_ref[...])
    s = r * t
    h = h_ref[...]
    b1 = b1_ref[...]
    b2 = b2_ref[...]
    out = (jnp.dot(a + h, w1_ref[...], preferred_element_type=jnp.float32)
           + jnp.dot(a * h, w2_ref[...], preferred_element_type=jnp.float32)
           + b1 + s * (b1 + b2))
    if leaky:
        out = jnp.where(out > 0, out, 0.01 * out)
    out_refs[0][...] = out
    if with_g:
        out_refs[1][...] = out
        out_refs[2][...] = r * out


def _tc_dense(ap, tp, r16, h, w1, b1, w2, b2, o384, leaky, with_g, col):
    out_specs = [pl.BlockSpec((BN, D), lambda i, c=col: (i, c))]
    out_shape = [jax.ShapeDtypeStruct((N, 3 * D), jnp.float32)]
    if with_g:
        out_specs.append(pl.BlockSpec((BN, D), lambda i: (i, 0)))
        out_shape.append(jax.ShapeDtypeStruct((NA, D), jnp.float32))
        out_specs.append(pl.BlockSpec((BN, D), lambda i: (i, 0)))
        out_shape.append(jax.ShapeDtypeStruct((NA, D), jnp.float32))
    return pl.pallas_call(
        functools.partial(_dense_kernel, leaky, with_g),
        grid=(GRID,),
        in_specs=[
            pl.BlockSpec((BN, D), lambda i: (i, 0)),
            pl.BlockSpec((BN, D), lambda i: (i + GRID, 0)),
            pl.BlockSpec((BN, L2), lambda i: (i, 0)),
            pl.BlockSpec((BN, L2), lambda i: (i + GRID, 0)),
            pl.BlockSpec((BN, L), lambda i: (i, 0)),
            pl.BlockSpec((BN, D), lambda i: (i, 0)),
            pl.BlockSpec((D, D), lambda i: (0, 0)),
            pl.BlockSpec((1, D), lambda i: (0, 0)),
            pl.BlockSpec((D, D), lambda i: (0, 0)),
            pl.BlockSpec((1, D), lambda i: (0, 0)),
            pl.BlockSpec(memory_space=pl.ANY),
        ],
        out_specs=out_specs,
        out_shape=out_shape,
        input_output_aliases={10: 0},
    )(ap, ap, tp, tp, r16, h, w1, b1, w2, b2, o384)


def kernel(features, edge_index, W1_0, b1_0, W2_0, b2_0, W1_1, b1_1, W2_1, b2_1):
    src = edge_index[0]
    dst = edge_index[1]
    zc = jnp.zeros((ROWS_W, D), jnp.float32)
    zt = jnp.zeros((ROWS_W * L2,), jnp.float32)

    degp, dstp = _sc_deg(src, dst, zt)
    g0, r16, o384 = _tc_prep(degp.reshape(NC * NA, L2), features)
    rflat = r16[:, 0]

    ap0, tp = _sc_gs(g0, src, dstp, rflat, zc, zt, with_t=True)
    tp = tp.reshape(NC * NA, L2)
    o384, h1, g1 = _tc_dense(ap0, tp, r16, features,
                             W1_0, b1_0.reshape(1, D), W2_0, b2_0.reshape(1, D),
                             o384, leaky=True, with_g=True, col=1)
    ap1, _unused = _sc_gs(g1, src, dstp, rflat, zc, zt, with_t=False)
    (o384,) = _tc_dense(ap1, tp, r16, h1,
                        W1_1, b1_1.reshape(1, D), W2_1, b2_1.reshape(1, D),
                        o384, leaky=False, with_g=False, col=2)

    return o384
